# Initial kernel scaffold; baseline (speedup 1.0000x reference)
import jax
import jax.numpy as jnp
from jax.experimental import pallas as pl


def kernel(x, table, W0, b0, W1, b1, W2, b2, W3, b3, W4, b4):
    raise NotImplementedError("write your pallas kernel here")


# speedup vs baseline: 1.2319x; 1.2319x over previous
import functools

import numpy as np
import jax
import jax.numpy as jnp
from jax import lax
from jax.experimental import pallas as pl
from jax.experimental.pallas import tpu as pltpu
from jax.experimental.pallas import tpu_sc as plsc

_NUM_LEVELS = 16
_BASE_RES = 16
_MAX_PARAMS = 2 ** 19
_DESIRED_RES = 2048
_N = 1048576
_PER_LEVEL_SCALE = float(np.exp2(np.log2(_DESIRED_RES / _BASE_RES) / (_NUM_LEVELS - 1)))
_P1 = int(np.uint32(2654435761).astype(np.int32))
_P2 = int(np.uint32(805459861).astype(np.int32))


def _levels():
    scales, resolutions, offsets, sizes = [], [], [], []
    offset = 0
    for l in range(_NUM_LEVELS):
        scale = _BASE_RES * (_PER_LEVEL_SCALE ** l) - 1.0
        res = int(np.ceil(scale)) + 1
        params = min(_MAX_PARAMS, res ** 3)
        params = int(np.ceil(params / 8) * 8)
        scales.append(scale)
        resolutions.append(res)
        offsets.append(offset)
        sizes.append(params)
        offset += params
    return scales, resolutions, offsets, sizes


_SCALES, _RES, _OFF, _SIZES = _levels()
_USE_HASH = [(r ** 3) > s for r, s in zip(_RES, _SIZES)]

_NC, _NS = 2, 16
_NW = _NC * _NS
_C = 512
_NPW = _N // _NW
_NCHUNK = _NPW // _C


def _enc_body(x0h, x1h, x2h, tabh, feath, xv0, xv1, xv2,
              idxv0, idxv1, rowsv0, rowsv1, featv, sem0, sem1):
    wid = lax.axis_index("s") * _NC + lax.axis_index("c")
    sems = (sem0, sem1)
    idxvs = (idxv0, idxv1)
    rowsvs = (rowsv0, rowsv1)
    iota = lax.iota(jnp.int32, 16)
    col0 = jnp.zeros((16,), jnp.int32)
    col1 = jnp.ones((16,), jnp.int32)

    def idx_phase(l, b):
        scale = jnp.float32(_SCALES[l])
        off = _OFF[l]
        idxv = idxvs[b]

        def jb(j, carry):
            o = pl.multiple_of(j * 16, 16)
            px = xv0[pl.ds(o, 16)] * scale + 0.5
            py = xv1[pl.ds(o, 16)] * scale + 0.5
            pz = xv2[pl.ds(o, 16)] * scale + 0.5
            gx = px.astype(jnp.int32)
            gy = py.astype(jnp.int32)
            gz = pz.astype(jnp.int32)
            gx1 = gx + 1
            gy1 = gy + 1
            gz1 = gz + 1
            if _USE_HASH[l]:
                m = _SIZES[l] - 1
                hy0 = gy * _P1
                hy1 = gy1 * _P1
                hz0 = gz * _P2
                hz1 = gz1 * _P2
                xy = (gx ^ hy0, gx1 ^ hy0, gx ^ hy1, gx1 ^ hy1)
                hz = (hz0, hz1)
                for c in range(8):
                    idx = ((xy[c & 3] ^ hz[c >> 2]) & m) + off
                    idxv[pl.ds(c * _C + o, 16)] = idx
            else:
                res = _RES[l]
                sy0 = gy * res
                sy1 = gy1 * res
                sz0 = gz * (res * res)
                sz1 = gz1 * (res * res)
                yz = (sy0 + sz0, sy1 + sz0, sy0 + sz1, sy1 + sz1)
                gxs = (gx, gx1)
                for c in range(8):
                    idx = gxs[c & 1] + yz[c >> 1] + off
                    idxv[pl.ds(c * _C + o, 16)] = idx
            return carry

        lax.fori_loop(0, _C // 16, jb, 0)

    def acc_phase(l, b):
        scale = jnp.float32(_SCALES[l])
        rows = rowsvs[b]
        cl0 = jnp.full((16,), 2 * l, jnp.int32)
        cl1 = jnp.full((16,), 2 * l + 1, jnp.int32)

        def jb(j, carry):
            o = pl.multiple_of(j * 16, 16)
            px = xv0[pl.ds(o, 16)] * scale + 0.5
            py = xv1[pl.ds(o, 16)] * scale + 0.5
            pz = xv2[pl.ds(o, 16)] * scale + 0.5
            fx = px - px.astype(jnp.int32).astype(jnp.float32)
            fy = py - py.astype(jnp.int32).astype(jnp.float32)
            fz = pz - pz.astype(jnp.int32).astype(jnp.float32)
            wx = (1.0 - fx, fx)
            wy = (1.0 - fy, fy)
            wz = (1.0 - fz, fz)
            wxy = (wx[0] * wy[0], wx[1] * wy[0], wx[0] * wy[1], wx[1] * wy[1])
            acc0 = None
            acc1 = None
            for c in range(8):
                w = wxy[c & 3] * wz[c >> 2]
                ridx = iota + (c * _C + o)
                r0 = plsc.load_gather(rows, [ridx, col0])
                r1 = plsc.load_gather(rows, [ridx, col1])
                if c == 0:
                    acc0 = w * r0
                    acc1 = w * r1
                else:
                    acc0 = acc0 + w * r0
                    acc1 = acc1 + w * r1
            prow = iota + o
            plsc.store_scatter(featv, [prow, cl0], acc0)
            plsc.store_scatter(featv, [prow, cl1], acc1)
            return carry

        lax.fori_loop(0, _C // 16, jb, 0)

    def chunk_body(ci, carry):
        base = pl.multiple_of(wid * _NPW + ci * _C, _C)
        pltpu.sync_copy(x0h.at[pl.ds(base, _C)], xv0)
        pltpu.sync_copy(x1h.at[pl.ds(base, _C)], xv1)
        pltpu.sync_copy(x2h.at[pl.ds(base, _C)], xv2)

        idx_phase(0, 0)
        handles = [None, None]
        handles[0] = pltpu.async_copy(tabh.at[idxvs[0]], rowsvs[0], sems[0])
        for l in range(1, _NUM_LEVELS):
            b = l % 2
            bp = (l - 1) % 2
            idx_phase(l, b)
            handles[b] = pltpu.async_copy(tabh.at[idxvs[b]], rowsvs[b], sems[b])
            handles[bp].wait()
            acc_phase(l - 1, bp)
        handles[(_NUM_LEVELS - 1) % 2].wait()
        acc_phase(_NUM_LEVELS - 1, (_NUM_LEVELS - 1) % 2)

        pltpu.sync_copy(featv, feath.at[pl.ds(base, _C)])
        return carry

    lax.fori_loop(0, _NCHUNK, chunk_body, 0)


@functools.partial(
    pl.kernel,
    out_type=jax.ShapeDtypeStruct((_N, 32), jnp.float32),
    mesh=plsc.VectorSubcoreMesh(core_axis_name="c", subcore_axis_name="s"),
    scratch_types=[
        pltpu.VMEM((_C,), jnp.float32),
        pltpu.VMEM((_C,), jnp.float32),
        pltpu.VMEM((_C,), jnp.float32),
        pltpu.VMEM((8 * _C,), jnp.int32),
        pltpu.VMEM((8 * _C,), jnp.int32),
        pltpu.VMEM((8 * _C, 2), jnp.float32),
        pltpu.VMEM((8 * _C, 2), jnp.float32),
        pltpu.VMEM((_C, 32), jnp.float32),
        pltpu.SemaphoreType.DMA,
        pltpu.SemaphoreType.DMA,
    ],
    compiler_params=pltpu.CompilerParams(
        needs_layout_passes=False, use_tc_tiling_on_sc=False),
)
def _encode(*args):
    _enc_body(*args)


_B = 4096


def _mlp_body(fref, w0r, w1r, w2r, w3r, w4r, b0r, b1r, b2r, b3r, b4r, oref):
    dn = (((1,), (1,)), ((), ()))
    h = fref[...]
    h = jnp.maximum(
        lax.dot_general(h, w0r[...], dn, preferred_element_type=jnp.float32)
        + b0r[...], 0.0)
    h = jnp.maximum(
        lax.dot_general(h, w1r[...], dn, preferred_element_type=jnp.float32)
        + b1r[...], 0.0)
    h = jnp.maximum(
        lax.dot_general(h, w2r[...], dn, preferred_element_type=jnp.float32)
        + b2r[...], 0.0)
    h = jnp.maximum(
        lax.dot_general(h, w3r[...], dn, preferred_element_type=jnp.float32)
        + b3r[...], 0.0)
    out8 = lax.dot_general(h, w4r[...], dn, preferred_element_type=jnp.float32)
    oref[...] = out8[:, 0:1] + b4r[0, 0]


def _full_spec(shape):
    nd = len(shape)
    return pl.BlockSpec(shape, lambda i: (0,) * nd)


def _mlp(feats, W0, W1, W2, W3, W4, b0, b1, b2, b3, b4):
    grid = (_N // _B,)
    return pl.pallas_call(
        _mlp_body,
        grid=grid,
        in_specs=[
            pl.BlockSpec((_B, 32), lambda i: (i, 0)),
            _full_spec(W0.shape), _full_spec(W1.shape), _full_spec(W2.shape),
            _full_spec(W3.shape), _full_spec(W4.shape),
            _full_spec(b0.shape), _full_spec(b1.shape), _full_spec(b2.shape),
            _full_spec(b3.shape),
            pl.BlockSpec(memory_space=pltpu.SMEM),
        ],
        out_specs=pl.BlockSpec((_B, 1), lambda i: (i, 0)),
        out_shape=jax.ShapeDtypeStruct((_N, 1), jnp.float32),
    )(feats, W0, W1, W2, W3, W4, b0, b1, b2, b3, b4)


def kernel(x, table, W0, b0, W1, b1, W2, b2, W3, b3, W4, b4):
    x0 = x[:, 0]
    x1 = x[:, 1]
    x2 = x[:, 2]
    feats = _encode(x0, x1, x2, table)
    W4p = jnp.pad(W4, ((0, 7), (0, 0)))
    return _mlp(
        feats, W0, W1, W2, W3, W4p,
        b0.reshape(1, -1), b1.reshape(1, -1), b2.reshape(1, -1),
        b3.reshape(1, -1), b4.reshape(1, 1),
    )


# speedup vs baseline: 1.3848x; 1.1241x over previous
import functools

import numpy as np
import jax
import jax.numpy as jnp
from jax import lax
from jax.experimental import pallas as pl
from jax.experimental.pallas import tpu as pltpu
from jax.experimental.pallas import tpu_sc as plsc

_NUM_LEVELS = 16
_BASE_RES = 16
_MAX_PARAMS = 2 ** 19
_DESIRED_RES = 2048
_N = 1048576
_PER_LEVEL_SCALE = float(np.exp2(np.log2(_DESIRED_RES / _BASE_RES) / (_NUM_LEVELS - 1)))
_P1 = int(np.uint32(2654435761).astype(np.int32))
_P2 = int(np.uint32(805459861).astype(np.int32))


def _levels():
    scales, resolutions, offsets, sizes = [], [], [], []
    offset = 0
    for l in range(_NUM_LEVELS):
        scale = _BASE_RES * (_PER_LEVEL_SCALE ** l) - 1.0
        res = int(np.ceil(scale)) + 1
        params = min(_MAX_PARAMS, res ** 3)
        params = int(np.ceil(params / 8) * 8)
        scales.append(scale)
        resolutions.append(res)
        offsets.append(offset)
        sizes.append(params)
        offset += params
    return scales, resolutions, offsets, sizes


_SCALES, _RES, _OFF, _SIZES = _levels()
_USE_HASH = [(r ** 3) > s for r, s in zip(_RES, _SIZES)]

_NC, _NS = 2, 16
_NW = _NC * _NS
_C = 512
_NPW = _N // _NW
_NCHUNK = _NPW // _C


def _enc_body(x0h, x1h, x2h, tabh, feath, xv0, xv1, xv2,
              idxv0, idxv1, colv0, colv1, rowsv0, rowsv1, featv, sem0, sem1):
    wid = lax.axis_index("s") * _NC + lax.axis_index("c")
    sems = (sem0, sem1)
    idxvs = (idxv0, idxv1)
    colvs = (colv0, colv1)
    rowsvs = (rowsv0, rowsv1)
    iota = lax.iota(jnp.int32, 16)

    def idx_phase(l, b):
        scale = jnp.float32(_SCALES[l])
        off = _OFF[l]
        idxv = idxvs[b]
        colv = colvs[b]

        def jb(j, carry):
            o = pl.multiple_of(j * 16, 16)
            px = xv0[pl.ds(o, 16)] * scale + 0.5
            py = xv1[pl.ds(o, 16)] * scale + 0.5
            pz = xv2[pl.ds(o, 16)] * scale + 0.5
            gx = px.astype(jnp.int32)
            gy = py.astype(jnp.int32)
            gz = pz.astype(jnp.int32)
            gx1 = gx + 1
            gy1 = gy + 1
            gz1 = gz + 1
            if _USE_HASH[l]:
                m = _SIZES[l] - 1
                hy0 = gy * _P1
                hy1 = gy1 * _P1
                hz0 = gz * _P2
                hz1 = gz1 * _P2
                xy = (gx ^ hy0, gx1 ^ hy0, gx ^ hy1, gx1 ^ hy1)
                hz = (hz0, hz1)
                for c in range(8):
                    idx = ((xy[c & 3] ^ hz[c >> 2]) & m) + off
                    idxv[pl.ds(c * _C + o, 16)] = idx >> 2
                    colv[pl.ds(c * _C + o, 16)] = (idx & 3) << 1
            else:
                res = _RES[l]
                sy0 = gy * res
                sy1 = gy1 * res
                sz0 = gz * (res * res)
                sz1 = gz1 * (res * res)
                yz = (sy0 + sz0, sy1 + sz0, sy0 + sz1, sy1 + sz1)
                gxs = (gx, gx1)
                for c in range(8):
                    idx = gxs[c & 1] + yz[c >> 1] + off
                    idxv[pl.ds(c * _C + o, 16)] = idx >> 2
                    colv[pl.ds(c * _C + o, 16)] = (idx & 3) << 1
            return carry

        lax.fori_loop(0, _C // 16, jb, 0)

    def acc_phase(l, b):
        scale = jnp.float32(_SCALES[l])
        rows = rowsvs[b]
        colv = colvs[b]
        cl0 = jnp.full((16,), 2 * l, jnp.int32)
        cl1 = jnp.full((16,), 2 * l + 1, jnp.int32)

        def jb(j, carry):
            o = pl.multiple_of(j * 16, 16)
            px = xv0[pl.ds(o, 16)] * scale + 0.5
            py = xv1[pl.ds(o, 16)] * scale + 0.5
            pz = xv2[pl.ds(o, 16)] * scale + 0.5
            fx = px - px.astype(jnp.int32).astype(jnp.float32)
            fy = py - py.astype(jnp.int32).astype(jnp.float32)
            fz = pz - pz.astype(jnp.int32).astype(jnp.float32)
            wx = (1.0 - fx, fx)
            wy = (1.0 - fy, fy)
            wz = (1.0 - fz, fz)
            wxy = (wx[0] * wy[0], wx[1] * wy[0], wx[0] * wy[1], wx[1] * wy[1])
            acc0 = None
            acc1 = None
            for c in range(8):
                w = wxy[c & 3] * wz[c >> 2]
                ridx = iota + (c * _C + o)
                cbase = colv[pl.ds(c * _C + o, 16)]
                r0 = plsc.load_gather(rows, [ridx, cbase])
                r1 = plsc.load_gather(rows, [ridx, cbase + 1])
                if c == 0:
                    acc0 = w * r0
                    acc1 = w * r1
                else:
                    acc0 = acc0 + w * r0
                    acc1 = acc1 + w * r1
            prow = iota + o
            plsc.store_scatter(featv, [prow, cl0], acc0)
            plsc.store_scatter(featv, [prow, cl1], acc1)
            return carry

        lax.fori_loop(0, _C // 16, jb, 0)

    def chunk_body(ci, carry):
        base = pl.multiple_of(wid * _NPW + ci * _C, _C)
        pltpu.sync_copy(x0h.at[pl.ds(base, _C)], xv0)
        pltpu.sync_copy(x1h.at[pl.ds(base, _C)], xv1)
        pltpu.sync_copy(x2h.at[pl.ds(base, _C)], xv2)

        idx_phase(0, 0)
        handles = [None, None]
        handles[0] = pltpu.async_copy(tabh.at[idxvs[0]], rowsvs[0], sems[0])
        for l in range(1, _NUM_LEVELS):
            b = l % 2
            bp = (l - 1) % 2
            idx_phase(l, b)
            handles[b] = pltpu.async_copy(tabh.at[idxvs[b]], rowsvs[b], sems[b])
            handles[bp].wait()
            acc_phase(l - 1, bp)
        handles[(_NUM_LEVELS - 1) % 2].wait()
        acc_phase(_NUM_LEVELS - 1, (_NUM_LEVELS - 1) % 2)

        pltpu.sync_copy(featv, feath.at[pl.ds(base, _C)])
        return carry

    lax.fori_loop(0, _NCHUNK, chunk_body, 0)


@functools.partial(
    pl.kernel,
    out_type=jax.ShapeDtypeStruct((_N, 32), jnp.float32),
    mesh=plsc.VectorSubcoreMesh(core_axis_name="c", subcore_axis_name="s"),
    scratch_types=[
        pltpu.VMEM((_C,), jnp.float32),
        pltpu.VMEM((_C,), jnp.float32),
        pltpu.VMEM((_C,), jnp.float32),
        pltpu.VMEM((8 * _C,), jnp.int32),
        pltpu.VMEM((8 * _C,), jnp.int32),
        pltpu.VMEM((8 * _C,), jnp.int32),
        pltpu.VMEM((8 * _C,), jnp.int32),
        pltpu.VMEM((8 * _C, 8), jnp.float32),
        pltpu.VMEM((8 * _C, 8), jnp.float32),
        pltpu.VMEM((_C, 32), jnp.float32),
        pltpu.SemaphoreType.DMA,
        pltpu.SemaphoreType.DMA,
    ],
    compiler_params=pltpu.CompilerParams(
        needs_layout_passes=False, use_tc_tiling_on_sc=False),
)
def _encode(*args):
    _enc_body(*args)


_B = 4096


def _mlp_body(fref, w0r, w1r, w2r, w3r, w4r, b0r, b1r, b2r, b3r, b4r, oref):
    dn = (((1,), (1,)), ((), ()))
    h = fref[...]
    h = jnp.maximum(
        lax.dot_general(h, w0r[...], dn, preferred_element_type=jnp.float32)
        + b0r[...], 0.0)
    h = jnp.maximum(
        lax.dot_general(h, w1r[...], dn, preferred_element_type=jnp.float32)
        + b1r[...], 0.0)
    h = jnp.maximum(
        lax.dot_general(h, w2r[...], dn, preferred_element_type=jnp.float32)
        + b2r[...], 0.0)
    h = jnp.maximum(
        lax.dot_general(h, w3r[...], dn, preferred_element_type=jnp.float32)
        + b3r[...], 0.0)
    out8 = lax.dot_general(h, w4r[...], dn, preferred_element_type=jnp.float32)
    oref[...] = out8[:, 0:1] + b4r[0, 0]


def _full_spec(shape):
    nd = len(shape)
    return pl.BlockSpec(shape, lambda i: (0,) * nd)


def _mlp(feats, W0, W1, W2, W3, W4, b0, b1, b2, b3, b4):
    grid = (_N // _B,)
    return pl.pallas_call(
        _mlp_body,
        grid=grid,
        in_specs=[
            pl.BlockSpec((_B, 32), lambda i: (i, 0)),
            _full_spec(W0.shape), _full_spec(W1.shape), _full_spec(W2.shape),
            _full_spec(W3.shape), _full_spec(W4.shape),
            _full_spec(b0.shape), _full_spec(b1.shape), _full_spec(b2.shape),
            _full_spec(b3.shape),
            pl.BlockSpec(memory_space=pltpu.SMEM),
        ],
        out_specs=pl.BlockSpec((_B, 1), lambda i: (i, 0)),
        out_shape=jax.ShapeDtypeStruct((_N, 1), jnp.float32),
    )(feats, W0, W1, W2, W3, W4, b0, b1, b2, b3, b4)


def kernel(x, table, W0, b0, W1, b1, W2, b2, W3, b3, W4, b4):
    x0 = x[:, 0]
    x1 = x[:, 1]
    x2 = x[:, 2]
    tab4 = table.reshape(-1, 8)
    feats = _encode(x0, x1, x2, tab4)
    W4p = jnp.pad(W4, ((0, 7), (0, 0)))
    return _mlp(
        feats, W0, W1, W2, W3, W4p,
        b0.reshape(1, -1), b1.reshape(1, -1), b2.reshape(1, -1),
        b3.reshape(1, -1), b4.reshape(1, 1),
    )
